# baseline (speedup 1.0000x reference)
import functools

import jax
import jax.numpy as jnp
from jax import lax
from jax.experimental import pallas as pl
from jax.experimental.pallas import tpu as pltpu
from jax.experimental.pallas import tpu_sc as plsc

NU = 25000
NI = 25000
E = 800000
D = 64
NB = 4096
ROWS = E // 128
ROWSP = 6272
NP = 25600
NPAD = 25088
NT = 16
NW = 32
NCHUNK = ROWSP // NT

_MESH = functools.partial(
    plsc.VectorSubcoreMesh, core_axis_name="c", subcore_axis_name="s")

_f32 = jnp.float32
_i32 = jnp.int32


def _sc_params():
    return pltpu.CompilerParams(needs_layout_passes=False,
                                use_tc_tiling_on_sc=False)


def _rsqrt_newton(x):
    i = lax.bitcast_convert_type(x, _i32)
    i = jnp.int32(0x5F3759DF) - lax.shift_right_logical(i, 1)
    y = lax.bitcast_convert_type(i, _f32)
    for _ in range(3):
        y = y * (1.5 - 0.5 * x * y * y)
    return y


def _splat(vec, lane):
    idx = jnp.broadcast_to(lane, (16,)).astype(_i32)[:, None]
    return lax.gather(
        vec, idx,
        lax.GatherDimensionNumbers(offset_dims=(), collapsed_slice_dims=(0,),
                                   start_index_map=(0,)),
        slice_sizes=(1,), mode=lax.GatherScatterMode.PROMISE_IN_BOUNDS)


def _perm(vec, idxvec):
    return lax.gather(
        vec, idxvec[:, None],
        lax.GatherDimensionNumbers(offset_dims=(), collapsed_slice_dims=(0,),
                                   start_index_map=(0,)),
        slice_sizes=(1,), mode=lax.GatherScatterMode.PROMISE_IN_BOUNDS)


def _hsum(v):
    lanes = lax.iota(_i32, 16)
    t = v
    for k in (1, 2, 4, 8):
        t = t + _perm(t, jnp.bitwise_xor(lanes, k))
    return t


def _proj_body(u_ref, i_ref, w1a_ref, w1b_ref, b1_ref, pu_ref, pi_ref):
    pu_ref[...] = jnp.dot(u_ref[...], w1a_ref[...],
                          preferred_element_type=_f32)
    pi_ref[...] = jnp.dot(i_ref[...], w1b_ref[...],
                          preferred_element_type=_f32) + b1_ref[...]


def _project(user_emb, item_emb, w1a, w1b, b1):
    blk = 1000
    return pl.pallas_call(
        _proj_body,
        grid=(NU // blk,),
        in_specs=[
            pl.BlockSpec((blk, D), lambda i: (i, 0)),
            pl.BlockSpec((blk, D), lambda i: (i, 0)),
            pl.BlockSpec((D, D), lambda i: (0, 0)),
            pl.BlockSpec((D, D), lambda i: (0, 0)),
            pl.BlockSpec((1, D), lambda i: (0, 0)),
        ],
        out_specs=[
            pl.BlockSpec((blk, D), lambda i: (i, 0)),
            pl.BlockSpec((blk, D), lambda i: (i, 0)),
        ],
        out_shape=[
            jax.ShapeDtypeStruct((NU, D), _f32),
            jax.ShapeDtypeStruct((NI, D), _f32),
        ],
    )(user_emb, item_emb, w1a, w1b, b1.reshape(1, D))


def _degree_body(uip_ref, iip_ref, uep_ref, iep_ref,
                 dinvu_ref, dinvi_ref, zsu_ref, zsi_ref,
                 idx_v, ones_v, buf_v, xbuf_v, hist_sh, sem):
    c = lax.axis_index("c")
    s = lax.axis_index("s")

    def zinit(j, _):
        buf_v[pl.ds(j * 16, 16)] = jnp.zeros((16,), _f32)
        return 0
    lax.fori_loop(0, 100, zinit, 0)
    for g in range(8):
        ones_v[pl.ds(g * 16, 16)] = jnp.ones((16,), _f32)
    pltpu.sync_copy(buf_v, hist_sh.at[pl.ds(s * 1600, 1600)])
    plsc.subcore_barrier()

    def hist_loop(src_ref):
        def body(i, _):
            r = (i * NT + s) * 8
            pltpu.sync_copy(src_ref.at[pl.ds(r, 8)], idx_v)
            cps = [pltpu.async_copy(ones_v, hist_sh.at[idx_v.at[j]], sem,
                                    add=True) for j in range(8)]
            for cp in cps:
                cp.wait()
            return 0
        lax.fori_loop(0, ROWSP // 8 // NT, body, 0)

    @pl.when(c == 0)
    def _():
        hist_loop(uip_ref)

    @pl.when(c == 1)
    def _():
        hist_loop(iip_ref)

    plsc.subcore_barrier()

    pltpu.sync_copy(hist_sh.at[pl.ds(s * 1600, 1600)], buf_v)

    @pl.when(s == 0)
    def _():
        pad_fix = jnp.where(lax.iota(_i32, 16) == 0,
                            jnp.float32((ROWSP - ROWS) * 128), 0.0)
        buf_v[pl.ds(0, 16)] = buf_v[pl.ds(0, 16)] - pad_fix

    def conv(j, _):
        x = buf_v[pl.ds(j * 16, 16)] + 1e-8
        buf_v[pl.ds(j * 16, 16)] = _rsqrt_newton(x)
        return 0
    lax.fori_loop(0, 100, conv, 0)

    def scale_ego(x_ref, z_ref, dinv_ref):
        pltpu.sync_copy(buf_v, dinv_ref.at[pl.ds(s * 1600, 1600)])

        def cbody(i, _):
            base = s * 1600 + i * 160
            pltpu.sync_copy(x_ref.at[pl.ds(base, 160)], xbuf_v)

            @plsc.parallel_loop(0, 10, 1, unroll=2)
            def gbody(g):
                dv = buf_v[pl.ds(i * 160 + g * 16, 16)]
                for l in range(16):
                    e = g * 16 + l
                    wl = _splat(dv, l)
                    for k in range(4):
                        xbuf_v[e, pl.ds(k * 16, 16)] = (
                            xbuf_v[e, pl.ds(k * 16, 16)] * wl)
            pltpu.sync_copy(xbuf_v, z_ref.at[pl.ds(base, 160)])
            return 0
        lax.fori_loop(0, 10, cbody, 0)

    @pl.when(c == 0)
    def _():
        scale_ego(uep_ref, zsu_ref, dinvu_ref)

    @pl.when(c == 1)
    def _():
        scale_ego(iep_ref, zsi_ref, dinvi_ref)


def _degree(uip, iip, uep, iep):
    return pl.kernel(
        _degree_body,
        out_type=[
            jax.ShapeDtypeStruct((NP,), _f32),
            jax.ShapeDtypeStruct((NP,), _f32),
            jax.ShapeDtypeStruct((NP, D), _f32),
            jax.ShapeDtypeStruct((NP, D), _f32),
        ],
        mesh=_MESH(),
        compiler_params=_sc_params(),
        scratch_types=[
            pltpu.VMEM((8, 128), _i32),
            pltpu.VMEM((128,), _f32),
            pltpu.VMEM((1600,), _f32),
            pltpu.VMEM((160, D), _f32),
            pltpu.VMEM_SHARED((NP,), _f32),
            pltpu.SemaphoreType.DMA,
        ],
    )(uip, iip, uep, iep)


def _gate_body(pu_ref, pi_ref, uip_ref, iip_ref, gbp_ref, w2_ref,
               gp_ref,
               si0, si1, di0, di1, gb0, gb1, wmb0, wmb1,
               pur0, pur1, pir0, pir1, w2_v,
               sem_ia, sem_ib, sem_ga, sem_gb):
    c = lax.axis_index("c")
    s = lax.axis_index("s")
    w = s * 2 + c

    pltpu.sync_copy(w2_ref, w2_v)
    w2b = [w2_v[pl.ds(k * 16, 16)] for k in range(4)]
    lane_eq = [lax.iota(_i32, 16) == l for l in range(16)]

    NCH = ROWSP // 2 // NW
    bufs = ((si0, di0, gb0, wmb0, pur0, pir0, sem_ia, sem_ga),
            (si1, di1, gb1, wmb1, pur1, pir1, sem_ib, sem_gb))

    def row_of(k):
        return 64 * k + 2 * w

    def idx_copies(k, b):
        si, di, gb, _, _, _, sem_i, _ = bufs[b]
        r = row_of(k)
        return (pltpu.make_async_copy(uip_ref.at[pl.ds(r, 2)], si, sem_i),
                pltpu.make_async_copy(iip_ref.at[pl.ds(r, 2)], di, sem_i),
                pltpu.make_async_copy(gbp_ref.at[pl.ds(r, 2)], gb, sem_i))

    def gather_copies(b):
        si, di, _, _, pur, pir, _, sem_g = bufs[b]
        return (pltpu.make_async_copy(pu_ref.at[si.at[0]],
                                      pur.at[pl.ds(0, 128)], sem_g),
                pltpu.make_async_copy(pu_ref.at[si.at[1]],
                                      pur.at[pl.ds(128, 128)], sem_g),
                pltpu.make_async_copy(pi_ref.at[di.at[0]],
                                      pir.at[pl.ds(0, 128)], sem_g),
                pltpu.make_async_copy(pi_ref.at[di.at[1]],
                                      pir.at[pl.ds(128, 128)], sem_g))

    def issue(copies):
        for cp in copies:
            cp.start()

    def drain(copies):
        for cp in copies:
            cp.wait()

    def compute(k, b):
        _, _, gb, wmb, pur, pir, _, _ = bufs[b]
        r = row_of(k)

        @plsc.parallel_loop(0, 16, 1, unroll=2)
        def gbody(g):
            lg = jnp.zeros((16,), _f32)
            for l in range(16):
                e = g * 16 + l
                p = []
                for kk in range(4):
                    z = (pur[e, pl.ds(kk * 16, 16)]
                         + pir[e, pl.ds(kk * 16, 16)])
                    p.append(jnp.maximum(z, 0.0) * w2b[kk])
                ssum = (p[0] + p[1]) + (p[2] + p[3])
                tot = _hsum(ssum)
                lg = jnp.where(lane_eq[l], tot, lg)
            half = g // 8
            base = pl.ds((g % 8) * 16, 16)
            gin = lg * 5.0 + gb[half, base]
            gate = 1.0 / (1.0 + jnp.exp(-gin)) + 0.5
            valid = jnp.broadcast_to(r + half < ROWS, (16,))
            wmb[half, base] = jnp.where(valid, gate, 0.0)
        pltpu.sync_copy(wmb, gp_ref.at[pl.ds(r, 2)])

    issue(idx_copies(0, 0))
    drain(idx_copies(0, 0))
    issue(gather_copies(0))
    issue(idx_copies(1, 1))

    def body(j, _):
        drain(idx_copies(2 * j + 1, 1))
        issue(gather_copies(1))
        drain(gather_copies(0))
        compute(2 * j, 0)

        @pl.when(j < NCH // 2 - 1)
        def _():
            issue(idx_copies(2 * j + 2, 0))

        @pl.when(j < NCH // 2 - 1)
        def _():
            drain(idx_copies(2 * j + 2, 0))
            issue(gather_copies(0))
        drain(gather_copies(1))
        compute(2 * j + 1, 1)

        @pl.when(j < NCH // 2 - 1)
        def _():
            issue(idx_copies(2 * j + 3, 1))
        return 0
    lax.fori_loop(0, NCH // 2, body, 0)


def _gate(pu, pi, uip, iip, gbp, w2):
    return pl.kernel(
        _gate_body,
        out_type=jax.ShapeDtypeStruct((ROWSP, 128), _f32),
        mesh=_MESH(),
        compiler_params=_sc_params(),
        scratch_types=[
            pltpu.VMEM((2, 128), _i32),
            pltpu.VMEM((2, 128), _i32),
            pltpu.VMEM((2, 128), _i32),
            pltpu.VMEM((2, 128), _i32),
            pltpu.VMEM((2, 128), _f32),
            pltpu.VMEM((2, 128), _f32),
            pltpu.VMEM((2, 128), _f32),
            pltpu.VMEM((2, 128), _f32),
            pltpu.VMEM((256, D), _f32),
            pltpu.VMEM((256, D), _f32),
            pltpu.VMEM((256, D), _f32),
            pltpu.VMEM((256, D), _f32),
            pltpu.VMEM((64,), _f32),
            pltpu.SemaphoreType.DMA,
            pltpu.SemaphoreType.DMA,
            pltpu.SemaphoreType.DMA,
            pltpu.SemaphoreType.DMA,
        ],
    )(pu, pi, uip, iip, gbp, w2)


def _prop_body(zu_ref, zi_ref, uip_ref, iip_ref, gp_ref,
               dinvu_ref, dinvi_ref,
               yu_ref, yi_ref, nzu_ref, nzi_ref,
               si0, si1, di0, di1, wm0, wm1, rows0, rows1, dsl_v,
               acc_sh, sem_ia, sem_ib, sem_ga, sem_gb):
    c = lax.axis_index("c")
    s = lax.axis_index("s")

    def zinit(e, _):
        for k in range(4):
            rows0[e, pl.ds(k * 16, 16)] = jnp.zeros((16,), _f32)
        return 0
    lax.fori_loop(0, 128, zinit, 0)
    for k in range(12):
        pltpu.sync_copy(rows0, acc_sh.at[pl.ds(s * 1568 + k * 128, 128)])
    pltpu.sync_copy(rows0.at[pl.ds(0, 32)],
                    acc_sh.at[pl.ds(s * 1568 + 1536, 32)])
    plsc.subcore_barrier()

    def edge_loop(src2_ref, dst2_ref, xsrc_ref):
        bufs = ((si0, di0, wm0, rows0, sem_ia, sem_ga),
                (si1, di1, wm1, rows1, sem_ib, sem_gb))

        def row_of(k):
            return 16 * k + s

        def idx_copies(k, b):
            si, di, wm, _, sem_i, _ = bufs[b]
            r = row_of(k)
            return (pltpu.make_async_copy(src2_ref.at[r], si, sem_i),
                    pltpu.make_async_copy(dst2_ref.at[r], di, sem_i),
                    pltpu.make_async_copy(gp_ref.at[r], wm, sem_i))

        def gather_copies(b):
            si, _, _, rows, _, sem_g = bufs[b]
            return (pltpu.make_async_copy(xsrc_ref.at[si], rows, sem_g),)

        def issue(copies):
            for cp in copies:
                cp.start()

        def drain(copies):
            for cp in copies:
                cp.wait()

        def scale_scatter(b):
            _, di, wm, rows, _, _ = bufs[b]

            @plsc.parallel_loop(0, 8, 1, unroll=4)
            def gbody(g):
                wvec = wm[pl.ds(g * 16, 16)]
                for l in range(16):
                    e = g * 16 + l
                    wl = _splat(wvec, l)
                    for k in range(4):
                        rows[e, pl.ds(k * 16, 16)] = (
                            rows[e, pl.ds(k * 16, 16)] * wl)
            pltpu.sync_copy(rows, acc_sh.at[di], add=True)

        issue(idx_copies(0, 0))
        drain(idx_copies(0, 0))
        issue(gather_copies(0))
        issue(idx_copies(1, 1))

        def body(j, _):
            drain(idx_copies(2 * j + 1, 1))
            issue(gather_copies(1))
            drain(gather_copies(0))
            scale_scatter(0)

            @pl.when(j < NCHUNK // 2 - 1)
            def _():
                issue(idx_copies(2 * j + 2, 0))

            @pl.when(j < NCHUNK // 2 - 1)
            def _():
                drain(idx_copies(2 * j + 2, 0))
                issue(gather_copies(0))
            drain(gather_copies(1))
            scale_scatter(1)

            @pl.when(j < NCHUNK // 2 - 1)
            def _():
                issue(idx_copies(2 * j + 3, 1))
            return 0
        lax.fori_loop(0, NCHUNK // 2, body, 0)

    @pl.when(c == 0)
    def _():
        edge_loop(iip_ref, uip_ref, zi_ref)

    @pl.when(c == 1)
    def _():
        edge_loop(uip_ref, iip_ref, zu_ref)

    plsc.subcore_barrier()

    def drain_acc(y_ref, nz_ref, dinv_ref):
        pltpu.sync_copy(dinv_ref.at[pl.ds(s * 1568, 1568)], dsl_v)

        def scale_rows(k, ngrp):
            @plsc.parallel_loop(0, ngrp, 1, unroll=2)
            def gbody(g):
                dv = dsl_v[pl.ds(k * 128 + g * 16, 16)]
                for l in range(16):
                    e = g * 16 + l
                    wl = _splat(dv, l)
                    for kk in range(4):
                        rows0[e, pl.ds(kk * 16, 16)] = (
                            rows0[e, pl.ds(kk * 16, 16)] * wl)

        def cbody(k, _):
            base = s * 1568 + k * 128
            pltpu.sync_copy(acc_sh.at[pl.ds(base, 128)], rows0)
            pltpu.sync_copy(rows0, y_ref.at[pl.ds(base, 128)])
            scale_rows(k, 8)
            pltpu.sync_copy(rows0, nz_ref.at[pl.ds(base, 128)])
            return 0
        lax.fori_loop(0, 12, cbody, 0)

        base = s * 1568 + 1536
        tail = rows0.at[pl.ds(0, 32)]
        pltpu.sync_copy(acc_sh.at[pl.ds(base, 32)], tail)
        pltpu.sync_copy(tail, y_ref.at[pl.ds(base, 32)])
        scale_rows(12, 2)
        pltpu.sync_copy(tail, nz_ref.at[pl.ds(base, 32)])

    @pl.when(c == 0)
    def _():
        drain_acc(yu_ref, nzu_ref, dinvu_ref)

    @pl.when(c == 1)
    def _():
        drain_acc(yi_ref, nzi_ref, dinvi_ref)


def _propagate(zu, zi, uip, iip, gp, dinvu, dinvi):
    return pl.kernel(
        _prop_body,
        out_type=[
            jax.ShapeDtypeStruct((NPAD, D), _f32),
            jax.ShapeDtypeStruct((NPAD, D), _f32),
            jax.ShapeDtypeStruct((NPAD, D), _f32),
            jax.ShapeDtypeStruct((NPAD, D), _f32),
        ],
        mesh=_MESH(),
        compiler_params=_sc_params(),
        scratch_types=[
            pltpu.VMEM((128,), _i32),
            pltpu.VMEM((128,), _i32),
            pltpu.VMEM((128,), _i32),
            pltpu.VMEM((128,), _i32),
            pltpu.VMEM((128,), _f32),
            pltpu.VMEM((128,), _f32),
            pltpu.VMEM((128, D), _f32),
            pltpu.VMEM((128, D), _f32),
            pltpu.VMEM((1568,), _f32),
            pltpu.VMEM_SHARED((NPAD, D), _f32),
            pltpu.SemaphoreType.DMA,
            pltpu.SemaphoreType.DMA,
            pltpu.SemaphoreType.DMA,
            pltpu.SemaphoreType.DMA,
        ],
    )(zu, zi, uip, iip, gp, dinvu, dinvi)


def _predict_body(uid2_ref, iid2_ref,
                  x0u_ref, x1u_ref, x2u_ref, x3u_ref,
                  x0i_ref, x1i_ref, x2i_ref, x3i_ref,
                  preds2_ref,
                  uid_v, iid_v, su_v, si_v, out_v, sem):
    c = lax.axis_index("c")
    s = lax.axis_index("s")
    w = s * 2 + c

    pltpu.sync_copy(uid2_ref.at[w], uid_v)
    pltpu.sync_copy(iid2_ref.at[w], iid_v)
    pltpu.async_copy(x0u_ref.at[uid_v], su_v, sem).wait()
    for ref in (x1u_ref, x2u_ref, x3u_ref):
        pltpu.async_copy(ref.at[uid_v], su_v, sem, add=True).wait()
    pltpu.async_copy(x0i_ref.at[iid_v], si_v, sem).wait()
    for ref in (x1i_ref, x2i_ref, x3i_ref):
        pltpu.async_copy(ref.at[iid_v], si_v, sem, add=True).wait()

    lane_eq = [lax.iota(_i32, 16) == l for l in range(16)]

    @plsc.parallel_loop(0, 8, 1, unroll=2)
    def gbody(g):
        acc = jnp.zeros((16,), _f32)
        for l in range(16):
            e = g * 16 + l
            p = []
            for k in range(4):
                p.append(su_v[e, pl.ds(k * 16, 16)]
                         * si_v[e, pl.ds(k * 16, 16)])
            ssum = (p[0] + p[1]) + (p[2] + p[3])
            tot = _hsum(ssum)
            acc = jnp.where(lane_eq[l], tot, acc)
        out_v[pl.ds(g * 16, 16)] = acc * (1.0 / 16.0)
    pltpu.sync_copy(out_v, preds2_ref.at[w])


def _predict(uid2, iid2, xus, xis):
    return pl.kernel(
        _predict_body,
        out_type=jax.ShapeDtypeStruct((NW, 128), _f32),
        mesh=_MESH(),
        compiler_params=_sc_params(),
        scratch_types=[
            pltpu.VMEM((128,), _i32),
            pltpu.VMEM((128,), _i32),
            pltpu.VMEM((128, D), _f32),
            pltpu.VMEM((128, D), _f32),
            pltpu.VMEM((128,), _f32),
            pltpu.SemaphoreType.DMA,
        ],
    )(uid2, iid2, *xus, *xis)


def kernel(user_ids, item_ids, ui, ii, user_emb, item_emb, W1, b1, W2, b2):
    ui2 = ui.astype(_i32).reshape(ROWS, 128)
    ii2 = ii.astype(_i32).reshape(ROWS, 128)

    eps = jax.random.uniform(jax.random.key(42), (E,), dtype=_f32)
    gb2 = ((jnp.log(eps + 1e-8) - jnp.log(1.0 - eps + 1e-8) + b2[0])
           * 5.0).reshape(ROWS, 128)

    pad = ((0, ROWSP - ROWS), (0, 0))
    uip = jnp.pad(ui2, pad)
    iip = jnp.pad(ii2, pad)
    gbp = jnp.pad(gb2, pad)
    npad = ((0, NP - NU), (0, 0))
    uep = jnp.pad(user_emb, npad)
    iep = jnp.pad(item_emb, npad)

    pu, pi = _project(user_emb, item_emb, W1[:D], W1[D:], b1)
    dinvu, dinvi, zsu, zsi = _degree(uip, iip, uep, iep)
    gp = _gate(pu, pi, uip, iip, gbp, W2[:, 0])

    xus = [user_emb]
    xis = [item_emb]
    zu, zi = zsu, zsi
    for _ in range(3):
        yu, yi, zu, zi = _propagate(zu, zi, uip, iip, gp, dinvu, dinvi)
        xus.append(yu)
        xis.append(yi)

    uid2 = user_ids.astype(_i32).reshape(NW, 128)
    iid2 = item_ids.astype(_i32).reshape(NW, 128)
    preds2 = _predict(uid2, iid2, xus, xis)
    return preds2.reshape(NB)


# speedup vs baseline: 1.0032x; 1.0032x over previous
import functools

import jax
import jax.numpy as jnp
from jax import lax
from jax.experimental import pallas as pl
from jax.experimental.pallas import tpu as pltpu
from jax.experimental.pallas import tpu_sc as plsc

NU = 25000
NI = 25000
E = 800000
D = 64
NB = 4096
ROWS = E // 128
ROWSP = 6272
NP = 25600
NPAD = 25088
NT = 16
NW = 32
NCHUNK = ROWSP // NT

_MESH = functools.partial(
    plsc.VectorSubcoreMesh, core_axis_name="c", subcore_axis_name="s")

_f32 = jnp.float32
_i32 = jnp.int32


def _sc_params():
    return pltpu.CompilerParams(needs_layout_passes=False,
                                use_tc_tiling_on_sc=False)


def _rsqrt_newton(x):
    i = lax.bitcast_convert_type(x, _i32)
    i = jnp.int32(0x5F3759DF) - lax.shift_right_logical(i, 1)
    y = lax.bitcast_convert_type(i, _f32)
    for _ in range(3):
        y = y * (1.5 - 0.5 * x * y * y)
    return y


def _splat(vec, lane):
    idx = jnp.broadcast_to(lane, (16,)).astype(_i32)[:, None]
    return lax.gather(
        vec, idx,
        lax.GatherDimensionNumbers(offset_dims=(), collapsed_slice_dims=(0,),
                                   start_index_map=(0,)),
        slice_sizes=(1,), mode=lax.GatherScatterMode.PROMISE_IN_BOUNDS)


def _perm(vec, idxvec):
    return lax.gather(
        vec, idxvec[:, None],
        lax.GatherDimensionNumbers(offset_dims=(), collapsed_slice_dims=(0,),
                                   start_index_map=(0,)),
        slice_sizes=(1,), mode=lax.GatherScatterMode.PROMISE_IN_BOUNDS)


def _hsum(v):
    lanes = lax.iota(_i32, 16)
    t = v
    for k in (1, 2, 4, 8):
        t = t + _perm(t, jnp.bitwise_xor(lanes, k))
    return t


def _proj_body(u_ref, i_ref, w1a_ref, w1b_ref, b1_ref, pu_ref, pi_ref):
    pu_ref[...] = jnp.dot(u_ref[...], w1a_ref[...],
                          preferred_element_type=_f32)
    pi_ref[...] = jnp.dot(i_ref[...], w1b_ref[...],
                          preferred_element_type=_f32) + b1_ref[...]


def _project(user_emb, item_emb, w1a, w1b, b1):
    blk = 1000
    return pl.pallas_call(
        _proj_body,
        grid=(NU // blk,),
        in_specs=[
            pl.BlockSpec((blk, D), lambda i: (i, 0)),
            pl.BlockSpec((blk, D), lambda i: (i, 0)),
            pl.BlockSpec((D, D), lambda i: (0, 0)),
            pl.BlockSpec((D, D), lambda i: (0, 0)),
            pl.BlockSpec((1, D), lambda i: (0, 0)),
        ],
        out_specs=[
            pl.BlockSpec((blk, D), lambda i: (i, 0)),
            pl.BlockSpec((blk, D), lambda i: (i, 0)),
        ],
        out_shape=[
            jax.ShapeDtypeStruct((NU, D), _f32),
            jax.ShapeDtypeStruct((NI, D), _f32),
        ],
    )(user_emb, item_emb, w1a, w1b, b1.reshape(1, D))


def _degree_body(uip_ref, iip_ref, uep_ref, iep_ref,
                 dinvu_ref, dinvi_ref, zsu_ref, zsi_ref,
                 idx_v, ones_v, buf_v, xbuf_v, hist_sh, sem):
    c = lax.axis_index("c")
    s = lax.axis_index("s")

    def zinit(j, _):
        buf_v[pl.ds(j * 16, 16)] = jnp.zeros((16,), _f32)
        return 0
    lax.fori_loop(0, 100, zinit, 0)
    for g in range(8):
        ones_v[pl.ds(g * 16, 16)] = jnp.ones((16,), _f32)
    pltpu.sync_copy(buf_v, hist_sh.at[pl.ds(s * 1600, 1600)])
    plsc.subcore_barrier()

    def hist_loop(src_ref):
        def body(i, _):
            r = (i * NT + s) * 8
            pltpu.sync_copy(src_ref.at[pl.ds(r, 8)], idx_v)
            cps = [pltpu.async_copy(ones_v, hist_sh.at[idx_v.at[j]], sem,
                                    add=True) for j in range(8)]
            for cp in cps:
                cp.wait()
            return 0
        lax.fori_loop(0, ROWSP // 8 // NT, body, 0)

    @pl.when(c == 0)
    def _():
        hist_loop(uip_ref)

    @pl.when(c == 1)
    def _():
        hist_loop(iip_ref)

    plsc.subcore_barrier()

    pltpu.sync_copy(hist_sh.at[pl.ds(s * 1600, 1600)], buf_v)

    @pl.when(s == 0)
    def _():
        pad_fix = jnp.where(lax.iota(_i32, 16) == 0,
                            jnp.float32((ROWSP - ROWS) * 128), 0.0)
        buf_v[pl.ds(0, 16)] = buf_v[pl.ds(0, 16)] - pad_fix

    def conv(j, _):
        x = buf_v[pl.ds(j * 16, 16)] + 1e-8
        buf_v[pl.ds(j * 16, 16)] = _rsqrt_newton(x)
        return 0
    lax.fori_loop(0, 100, conv, 0)

    def scale_ego(x_ref, z_ref, dinv_ref):
        pltpu.sync_copy(buf_v, dinv_ref.at[pl.ds(s * 1600, 1600)])

        def cbody(i, _):
            base = s * 1600 + i * 160
            pltpu.sync_copy(x_ref.at[pl.ds(base, 160)], xbuf_v)

            @plsc.parallel_loop(0, 10, 1, unroll=2)
            def gbody(g):
                dv = buf_v[pl.ds(i * 160 + g * 16, 16)]
                for l in range(16):
                    e = g * 16 + l
                    wl = _splat(dv, l)
                    for k in range(4):
                        xbuf_v[e, pl.ds(k * 16, 16)] = (
                            xbuf_v[e, pl.ds(k * 16, 16)] * wl)
            pltpu.sync_copy(xbuf_v, z_ref.at[pl.ds(base, 160)])
            return 0
        lax.fori_loop(0, 10, cbody, 0)

    @pl.when(c == 0)
    def _():
        scale_ego(uep_ref, zsu_ref, dinvu_ref)

    @pl.when(c == 1)
    def _():
        scale_ego(iep_ref, zsi_ref, dinvi_ref)


def _degree(uip, iip, uep, iep):
    return pl.kernel(
        _degree_body,
        out_type=[
            jax.ShapeDtypeStruct((NP,), _f32),
            jax.ShapeDtypeStruct((NP,), _f32),
            jax.ShapeDtypeStruct((NP, D), _f32),
            jax.ShapeDtypeStruct((NP, D), _f32),
        ],
        mesh=_MESH(),
        compiler_params=_sc_params(),
        scratch_types=[
            pltpu.VMEM((8, 128), _i32),
            pltpu.VMEM((128,), _f32),
            pltpu.VMEM((1600,), _f32),
            pltpu.VMEM((160, D), _f32),
            pltpu.VMEM_SHARED((NP,), _f32),
            pltpu.SemaphoreType.DMA,
        ],
    )(uip, iip, uep, iep)


def _gate_body(pu_ref, pi_ref, uip_ref, iip_ref, gbp_ref, w2_ref,
               gp_ref,
               si0, si1, di0, di1, gb0, gb1, wmb0, wmb1,
               pur0, pur1, pir0, pir1, w2_v,
               sem_ia, sem_ib, sem_ga, sem_gb):
    c = lax.axis_index("c")
    s = lax.axis_index("s")
    w = s * 2 + c

    pltpu.sync_copy(w2_ref, w2_v)
    w2b = [w2_v[pl.ds(k * 16, 16)] for k in range(4)]
    lane_eq = [lax.iota(_i32, 16) == l for l in range(16)]

    NCH = ROWSP // 2 // NW
    bufs = ((si0, di0, gb0, wmb0, pur0, pir0, sem_ia, sem_ga),
            (si1, di1, gb1, wmb1, pur1, pir1, sem_ib, sem_gb))

    def row_of(k):
        return 64 * k + 2 * w

    def idx_copies(k, b):
        si, di, gb, _, _, _, sem_i, _ = bufs[b]
        r = row_of(k)
        return (pltpu.make_async_copy(uip_ref.at[pl.ds(r, 2)], si, sem_i),
                pltpu.make_async_copy(iip_ref.at[pl.ds(r, 2)], di, sem_i),
                pltpu.make_async_copy(gbp_ref.at[pl.ds(r, 2)], gb, sem_i))

    def gather_copies(b):
        si, di, _, _, pur, pir, _, sem_g = bufs[b]
        return (pltpu.make_async_copy(pu_ref.at[si.at[0]],
                                      pur.at[pl.ds(0, 128)], sem_g),
                pltpu.make_async_copy(pu_ref.at[si.at[1]],
                                      pur.at[pl.ds(128, 128)], sem_g),
                pltpu.make_async_copy(pi_ref.at[di.at[0]],
                                      pir.at[pl.ds(0, 128)], sem_g),
                pltpu.make_async_copy(pi_ref.at[di.at[1]],
                                      pir.at[pl.ds(128, 128)], sem_g))

    def issue(copies):
        for cp in copies:
            cp.start()

    def drain(copies):
        for cp in copies:
            cp.wait()

    def compute(k, b):
        _, _, gb, wmb, pur, pir, _, _ = bufs[b]
        r = row_of(k)

        @plsc.parallel_loop(0, 16, 1, unroll=2)
        def gbody(g):
            lg = jnp.zeros((16,), _f32)
            for l in range(16):
                e = g * 16 + l
                p = []
                for kk in range(4):
                    z = (pur[e, pl.ds(kk * 16, 16)]
                         + pir[e, pl.ds(kk * 16, 16)])
                    p.append(jnp.maximum(z, 0.0) * w2b[kk])
                ssum = (p[0] + p[1]) + (p[2] + p[3])
                tot = _hsum(ssum)
                lg = jnp.where(lane_eq[l], tot, lg)
            half = g // 8
            base = pl.ds((g % 8) * 16, 16)
            gin = lg * 5.0 + gb[half, base]
            gate = 1.0 / (1.0 + jnp.exp(-gin)) + 0.5
            valid = jnp.broadcast_to(r + half < ROWS, (16,))
            wmb[half, base] = jnp.where(valid, gate, 0.0)
        pltpu.sync_copy(wmb, gp_ref.at[pl.ds(r, 2)])

    issue(idx_copies(0, 0))
    drain(idx_copies(0, 0))
    issue(gather_copies(0))
    issue(idx_copies(1, 1))

    def body(j, _):
        drain(idx_copies(2 * j + 1, 1))
        issue(gather_copies(1))
        drain(gather_copies(0))
        compute(2 * j, 0)

        @pl.when(j < NCH // 2 - 1)
        def _():
            issue(idx_copies(2 * j + 2, 0))

        @pl.when(j < NCH // 2 - 1)
        def _():
            drain(idx_copies(2 * j + 2, 0))
            issue(gather_copies(0))
        drain(gather_copies(1))
        compute(2 * j + 1, 1)

        @pl.when(j < NCH // 2 - 1)
        def _():
            issue(idx_copies(2 * j + 3, 1))
        return 0
    lax.fori_loop(0, NCH // 2, body, 0)


def _gate(pu, pi, uip, iip, gbp, w2):
    return pl.kernel(
        _gate_body,
        out_type=jax.ShapeDtypeStruct((ROWSP, 128), _f32),
        mesh=_MESH(),
        compiler_params=_sc_params(),
        scratch_types=[
            pltpu.VMEM((2, 128), _i32),
            pltpu.VMEM((2, 128), _i32),
            pltpu.VMEM((2, 128), _i32),
            pltpu.VMEM((2, 128), _i32),
            pltpu.VMEM((2, 128), _f32),
            pltpu.VMEM((2, 128), _f32),
            pltpu.VMEM((2, 128), _f32),
            pltpu.VMEM((2, 128), _f32),
            pltpu.VMEM((256, D), _f32),
            pltpu.VMEM((256, D), _f32),
            pltpu.VMEM((256, D), _f32),
            pltpu.VMEM((256, D), _f32),
            pltpu.VMEM((64,), _f32),
            pltpu.SemaphoreType.DMA,
            pltpu.SemaphoreType.DMA,
            pltpu.SemaphoreType.DMA,
            pltpu.SemaphoreType.DMA,
        ],
    )(pu, pi, uip, iip, gbp, w2)


def _prop_body(zu_ref, zi_ref, uip_ref, iip_ref, gp_ref,
               dinvu_ref, dinvi_ref,
               yu_ref, yi_ref, nzu_ref, nzi_ref,
               si0, si1, di0, di1, wm0, wm1, rows0, rows1, dsl_v,
               acc_sh, sem_ia, sem_ib, sem_ga, sem_gb):
    c = lax.axis_index("c")
    s = lax.axis_index("s")

    def zinit(e, _):
        for k in range(4):
            rows0[e, pl.ds(k * 16, 16)] = jnp.zeros((16,), _f32)
        return 0
    lax.fori_loop(0, 128, zinit, 0)
    for k in range(12):
        pltpu.sync_copy(rows0, acc_sh.at[pl.ds(s * 1568 + k * 128, 128)])
    pltpu.sync_copy(rows0.at[pl.ds(0, 32)],
                    acc_sh.at[pl.ds(s * 1568 + 1536, 32)])
    plsc.subcore_barrier()

    def edge_loop(src2_ref, dst2_ref, xsrc_ref):
        bufs = ((si0, di0, wm0, rows0, sem_ia, sem_ga),
                (si1, di1, wm1, rows1, sem_ib, sem_gb))

        def row_of(k):
            return 16 * k + s

        def idx_copies(k, b):
            si, di, wm, _, sem_i, _ = bufs[b]
            r = row_of(k)
            return (pltpu.make_async_copy(src2_ref.at[r], si, sem_i),
                    pltpu.make_async_copy(dst2_ref.at[r], di, sem_i),
                    pltpu.make_async_copy(gp_ref.at[r], wm, sem_i))

        def gather_copies(b):
            si, _, _, rows, _, sem_g = bufs[b]
            return (pltpu.make_async_copy(xsrc_ref.at[si], rows, sem_g),)

        def issue(copies):
            for cp in copies:
                cp.start()

        def drain(copies):
            for cp in copies:
                cp.wait()

        def scale_scatter(b):
            _, di, wm, rows, _, _ = bufs[b]

            @plsc.parallel_loop(0, 8, 1, unroll=2)
            def gbody(g):
                wvec = wm[pl.ds(g * 16, 16)]
                for l in range(16):
                    e = g * 16 + l
                    wl = _splat(wvec, l)
                    for k in range(4):
                        rows[e, pl.ds(k * 16, 16)] = (
                            rows[e, pl.ds(k * 16, 16)] * wl)
            pltpu.sync_copy(rows, acc_sh.at[di], add=True)

        issue(idx_copies(0, 0))
        drain(idx_copies(0, 0))
        issue(gather_copies(0))
        issue(idx_copies(1, 1))

        def body(j, _):
            drain(idx_copies(2 * j + 1, 1))
            issue(gather_copies(1))
            drain(gather_copies(0))
            scale_scatter(0)

            @pl.when(j < NCHUNK // 2 - 1)
            def _():
                issue(idx_copies(2 * j + 2, 0))

            @pl.when(j < NCHUNK // 2 - 1)
            def _():
                drain(idx_copies(2 * j + 2, 0))
                issue(gather_copies(0))
            drain(gather_copies(1))
            scale_scatter(1)

            @pl.when(j < NCHUNK // 2 - 1)
            def _():
                issue(idx_copies(2 * j + 3, 1))
            return 0
        lax.fori_loop(0, NCHUNK // 2, body, 0)

    @pl.when(c == 0)
    def _():
        edge_loop(iip_ref, uip_ref, zi_ref)

    @pl.when(c == 1)
    def _():
        edge_loop(uip_ref, iip_ref, zu_ref)

    plsc.subcore_barrier()

    def drain_acc(y_ref, nz_ref, dinv_ref):
        pltpu.sync_copy(dinv_ref.at[pl.ds(s * 1568, 1568)], dsl_v)

        def scale_rows(k, ngrp):
            @plsc.parallel_loop(0, ngrp, 1, unroll=2)
            def gbody(g):
                dv = dsl_v[pl.ds(k * 128 + g * 16, 16)]
                for l in range(16):
                    e = g * 16 + l
                    wl = _splat(dv, l)
                    for kk in range(4):
                        rows0[e, pl.ds(kk * 16, 16)] = (
                            rows0[e, pl.ds(kk * 16, 16)] * wl)

        def cbody(k, _):
            base = s * 1568 + k * 128
            pltpu.sync_copy(acc_sh.at[pl.ds(base, 128)], rows0)
            pltpu.sync_copy(rows0, y_ref.at[pl.ds(base, 128)])
            scale_rows(k, 8)
            pltpu.sync_copy(rows0, nz_ref.at[pl.ds(base, 128)])
            return 0
        lax.fori_loop(0, 12, cbody, 0)

        base = s * 1568 + 1536
        tail = rows0.at[pl.ds(0, 32)]
        pltpu.sync_copy(acc_sh.at[pl.ds(base, 32)], tail)
        pltpu.sync_copy(tail, y_ref.at[pl.ds(base, 32)])
        scale_rows(12, 2)
        pltpu.sync_copy(tail, nz_ref.at[pl.ds(base, 32)])

    @pl.when(c == 0)
    def _():
        drain_acc(yu_ref, nzu_ref, dinvu_ref)

    @pl.when(c == 1)
    def _():
        drain_acc(yi_ref, nzi_ref, dinvi_ref)


def _propagate(zu, zi, uip, iip, gp, dinvu, dinvi):
    return pl.kernel(
        _prop_body,
        out_type=[
            jax.ShapeDtypeStruct((NPAD, D), _f32),
            jax.ShapeDtypeStruct((NPAD, D), _f32),
            jax.ShapeDtypeStruct((NPAD, D), _f32),
            jax.ShapeDtypeStruct((NPAD, D), _f32),
        ],
        mesh=_MESH(),
        compiler_params=_sc_params(),
        scratch_types=[
            pltpu.VMEM((128,), _i32),
            pltpu.VMEM((128,), _i32),
            pltpu.VMEM((128,), _i32),
            pltpu.VMEM((128,), _i32),
            pltpu.VMEM((128,), _f32),
            pltpu.VMEM((128,), _f32),
            pltpu.VMEM((128, D), _f32),
            pltpu.VMEM((128, D), _f32),
            pltpu.VMEM((1568,), _f32),
            pltpu.VMEM_SHARED((NPAD, D), _f32),
            pltpu.SemaphoreType.DMA,
            pltpu.SemaphoreType.DMA,
            pltpu.SemaphoreType.DMA,
            pltpu.SemaphoreType.DMA,
        ],
    )(zu, zi, uip, iip, gp, dinvu, dinvi)


def _predict_body(uid2_ref, iid2_ref,
                  x0u_ref, x1u_ref, x2u_ref, x3u_ref,
                  x0i_ref, x1i_ref, x2i_ref, x3i_ref,
                  preds2_ref,
                  uid_v, iid_v, su_v, si_v, out_v, sem):
    c = lax.axis_index("c")
    s = lax.axis_index("s")
    w = s * 2 + c

    pltpu.sync_copy(uid2_ref.at[w], uid_v)
    pltpu.sync_copy(iid2_ref.at[w], iid_v)
    pltpu.async_copy(x0u_ref.at[uid_v], su_v, sem).wait()
    for ref in (x1u_ref, x2u_ref, x3u_ref):
        pltpu.async_copy(ref.at[uid_v], su_v, sem, add=True).wait()
    pltpu.async_copy(x0i_ref.at[iid_v], si_v, sem).wait()
    for ref in (x1i_ref, x2i_ref, x3i_ref):
        pltpu.async_copy(ref.at[iid_v], si_v, sem, add=True).wait()

    lane_eq = [lax.iota(_i32, 16) == l for l in range(16)]

    @plsc.parallel_loop(0, 8, 1, unroll=2)
    def gbody(g):
        acc = jnp.zeros((16,), _f32)
        for l in range(16):
            e = g * 16 + l
            p = []
            for k in range(4):
                p.append(su_v[e, pl.ds(k * 16, 16)]
                         * si_v[e, pl.ds(k * 16, 16)])
            ssum = (p[0] + p[1]) + (p[2] + p[3])
            tot = _hsum(ssum)
            acc = jnp.where(lane_eq[l], tot, acc)
        out_v[pl.ds(g * 16, 16)] = acc * (1.0 / 16.0)
    pltpu.sync_copy(out_v, preds2_ref.at[w])


def _predict(uid2, iid2, xus, xis):
    return pl.kernel(
        _predict_body,
        out_type=jax.ShapeDtypeStruct((NW, 128), _f32),
        mesh=_MESH(),
        compiler_params=_sc_params(),
        scratch_types=[
            pltpu.VMEM((128,), _i32),
            pltpu.VMEM((128,), _i32),
            pltpu.VMEM((128, D), _f32),
            pltpu.VMEM((128, D), _f32),
            pltpu.VMEM((128,), _f32),
            pltpu.SemaphoreType.DMA,
        ],
    )(uid2, iid2, *xus, *xis)


def kernel(user_ids, item_ids, ui, ii, user_emb, item_emb, W1, b1, W2, b2):
    ui2 = ui.astype(_i32).reshape(ROWS, 128)
    ii2 = ii.astype(_i32).reshape(ROWS, 128)

    eps = jax.random.uniform(jax.random.key(42), (E,), dtype=_f32)
    gb2 = ((jnp.log(eps + 1e-8) - jnp.log(1.0 - eps + 1e-8) + b2[0])
           * 5.0).reshape(ROWS, 128)

    pad = ((0, ROWSP - ROWS), (0, 0))
    uip = jnp.pad(ui2, pad)
    iip = jnp.pad(ii2, pad)
    gbp = jnp.pad(gb2, pad)
    npad = ((0, NP - NU), (0, 0))
    uep = jnp.pad(user_emb, npad)
    iep = jnp.pad(item_emb, npad)

    pu, pi = _project(user_emb, item_emb, W1[:D], W1[D:], b1)
    dinvu, dinvi, zsu, zsi = _degree(uip, iip, uep, iep)
    gp = _gate(pu, pi, uip, iip, gbp, W2[:, 0])

    xus = [user_emb]
    xis = [item_emb]
    zu, zi = zsu, zsi
    for _ in range(3):
        yu, yi, zu, zi = _propagate(zu, zi, uip, iip, gp, dinvu, dinvi)
        xus.append(yu)
        xis.append(yi)

    uid2 = user_ids.astype(_i32).reshape(NW, 128)
    iid2 = item_ids.astype(_i32).reshape(NW, 128)
    preds2 = _predict(uid2, iid2, xus, xis)
    return preds2.reshape(NB)
